# Initial kernel scaffold; baseline (speedup 1.0000x reference)
import jax
import jax.numpy as jnp
from jax.experimental import pallas as pl


def _damp(v_ref, g_ref, o_ref):
    o_ref[...] = -g_ref[0] * v_ref[...]


def kernel(x, v, edge_index, logc, logr_c, gamma):
    g = jnp.reshape(gamma, (1,))
    return pl.pallas_call(
        _damp,
        out_shape=jax.ShapeDtypeStruct(v.shape, v.dtype),
    )(v, g)


# speedup vs baseline: 51.6496x; 51.6496x over previous
import jax
import jax.numpy as jnp
from jax import lax
from jax.experimental import pallas as pl
from jax.experimental.pallas import tpu as pltpu
from jax.experimental.pallas import tpu_sc as plsc

N = 100000
NP = 100096
E = 6400000
NC = 2
NS = 16
NW = NC * NS
L = 16

CB = 2048
RC = CB // 128
CHUNKS = -(-(E // NW) // CB)
EW = CHUNKS * CB
E_PAD = NW * EW
ROWS_W = EW // 128
NPT = NP // NS

_CT = 9392
_FLEN = NW * _CT


def _edge_body(x0h, x1h, x2h, srows, drows, logc16, logrc16, p_out,
               x0, x1, x2, a0, a1, a2,
               sidx, didx, xs0, xs1, xs2, xd0, xd1, xd2, m0, m1, m2,
               stg, lcv, lrv, gsem, ssem):
    cid = lax.axis_index("c")
    sid = lax.axis_index("s")
    wid = sid * NC + cid

    sl = pl.ds(sid * NPT, NPT)
    zeros = jnp.zeros((L,), jnp.float32)

    def zstg(i, _):
        stg[pl.ds(i * L, L)] = zeros
        return 0
    lax.fori_loop(0, NPT // L, zstg, 0)
    pltpu.sync_copy(stg, a0.at[sl])
    pltpu.sync_copy(stg, a1.at[sl])
    pltpu.sync_copy(stg, a2.at[sl])
    pltpu.sync_copy(x0h.at[sl], stg)
    pltpu.sync_copy(stg, x0.at[sl])
    pltpu.sync_copy(x1h.at[sl], stg)
    pltpu.sync_copy(stg, x1.at[sl])
    pltpu.sync_copy(x2h.at[sl], stg)
    pltpu.sync_copy(stg, x2.at[sl])
    pltpu.sync_copy(logc16, lcv)
    pltpu.sync_copy(logrc16, lrv)
    plsc.subcore_barrier()

    c2 = -2.0 * jnp.exp(lcv[...])
    rc = jnp.exp(lrv[...])

    row0 = wid * ROWS_W

    def chunk(ci, _):
        r = row0 + ci * RC
        pltpu.sync_copy(srows.at[pl.ds(r, RC)], sidx)
        pltpu.sync_copy(drows.at[pl.ds(r, RC)], didx)
        for j in range(RC):
            d = pl.ds(j * 128, 128)
            cps = [
                pltpu.async_copy(x0.at[sidx.at[j]], xs0.at[d], gsem),
                pltpu.async_copy(x1.at[sidx.at[j]], xs1.at[d], gsem),
                pltpu.async_copy(x2.at[sidx.at[j]], xs2.at[d], gsem),
                pltpu.async_copy(x0.at[didx.at[j]], xd0.at[d], gsem),
                pltpu.async_copy(x1.at[didx.at[j]], xd1.at[d], gsem),
                pltpu.async_copy(x2.at[didx.at[j]], xd2.at[d], gsem),
            ]
            for cp in cps:
                cp.wait()

        def group(g, _):
            s = pl.ds(g * L, L)
            dr0 = xd0[s] - xs0[s]
            dr1 = xd1[s] - xs1[s]
            dr2 = xd2[s] - xs2[s]
            r2 = jnp.maximum(dr0 * dr0 + dr1 * dr1 + dr2 * dr2, 1e-26)
            t = r2 * jnp.float32(4.0 ** 45)
            sc = jnp.zeros((L,), jnp.float32) + jnp.float32(2.0 ** 45)
            for e in (32, 16, 8, 4, 2, 1):
                cnd = t >= jnp.float32(4.0 ** e)
                t = jnp.where(cnd, t * jnp.float32(4.0 ** -e), t)
                sc = jnp.where(cnd, sc * jnp.float32(2.0 ** -e), sc)
            y = 1.39518 + (-0.45231 + 0.05713 * t) * t
            y = y * (1.5 - 0.5 * t * y * y)
            y = y * (1.5 - 0.5 * t * y * y)
            y = y * (1.5 - 0.5 * t * y * y)
            y = sc * y
            absdr = r2 * y
            yc = jnp.minimum(y, 1e12)
            w = (c2 * (absdr - rc)) * yc
            m0[s] = w * dr0
            m1[s] = w * dr1
            m2[s] = w * dr2
            return 0
        lax.fori_loop(0, CB // L, group, 0)

        for j in range(RC):
            d = pl.ds(j * 128, 128)
            cps = [
                pltpu.async_copy(m0.at[d], a0.at[didx.at[j]], ssem, add=True),
                pltpu.async_copy(m1.at[d], a1.at[didx.at[j]], ssem, add=True),
                pltpu.async_copy(m2.at[d], a2.at[didx.at[j]], ssem, add=True),
            ]
            for cp in cps:
                cp.wait()
        return 0

    lax.fori_loop(0, CHUNKS, chunk, 0)
    plsc.subcore_barrier()
    base = cid * 3 * NP + sid * NPT
    pltpu.sync_copy(a0.at[sl], stg)
    pltpu.sync_copy(stg, p_out.at[pl.ds(base, NPT)])
    pltpu.sync_copy(a1.at[sl], stg)
    pltpu.sync_copy(stg, p_out.at[pl.ds(base + NP, NPT)])
    pltpu.sync_copy(a2.at[sl], stg)
    pltpu.sync_copy(stg, p_out.at[pl.ds(base + 2 * NP, NPT)])


def _combine_body(f0, f1, vf, g16, out, b0, b1, bv, gv):
    cid = lax.axis_index("c")
    sid = lax.axis_index("s")
    wid = sid * NC + cid
    base = wid * _CT
    pltpu.sync_copy(f0.at[pl.ds(base, _CT)], b0)
    pltpu.sync_copy(f1.at[pl.ds(base, _CT)], b1)
    pltpu.sync_copy(vf.at[pl.ds(base, _CT)], bv)
    pltpu.sync_copy(g16, gv)
    g = gv[...]

    def step(i, _):
        s = pl.ds(i * L, L)
        b0[s] = b0[s] + b1[s] - g * bv[s]
        return 0
    lax.fori_loop(0, _CT // L, step, 0)
    pltpu.sync_copy(b0, out.at[pl.ds(base, _CT)])


_MESH = plsc.VectorSubcoreMesh(core_axis_name="c", subcore_axis_name="s")

_edge_call = pl.kernel(
    _edge_body,
    out_type=jax.ShapeDtypeStruct((6 * NP,), jnp.float32),
    mesh=_MESH,
    scratch_types=[
        pltpu.VMEM_SHARED((NP,), jnp.float32),
        pltpu.VMEM_SHARED((NP,), jnp.float32),
        pltpu.VMEM_SHARED((NP,), jnp.float32),
        pltpu.VMEM_SHARED((NP,), jnp.float32),
        pltpu.VMEM_SHARED((NP,), jnp.float32),
        pltpu.VMEM_SHARED((NP,), jnp.float32),
        pltpu.VMEM((RC, 128), jnp.int32),
        pltpu.VMEM((RC, 128), jnp.int32),
        pltpu.VMEM((CB,), jnp.float32),
        pltpu.VMEM((CB,), jnp.float32),
        pltpu.VMEM((CB,), jnp.float32),
        pltpu.VMEM((CB,), jnp.float32),
        pltpu.VMEM((CB,), jnp.float32),
        pltpu.VMEM((CB,), jnp.float32),
        pltpu.VMEM((CB,), jnp.float32),
        pltpu.VMEM((CB,), jnp.float32),
        pltpu.VMEM((CB,), jnp.float32),
        pltpu.VMEM((NPT,), jnp.float32),
        pltpu.VMEM((L,), jnp.float32),
        pltpu.VMEM((L,), jnp.float32),
        pltpu.SemaphoreType.DMA,
        pltpu.SemaphoreType.DMA,
    ],
)

_combine_call = pl.kernel(
    _combine_body,
    out_type=jax.ShapeDtypeStruct((_FLEN,), jnp.float32),
    mesh=_MESH,
    scratch_types=[
        pltpu.VMEM((_CT,), jnp.float32),
        pltpu.VMEM((_CT,), jnp.float32),
        pltpu.VMEM((_CT,), jnp.float32),
        pltpu.VMEM((L,), jnp.float32),
    ],
)


@jax.jit
def kernel(x, v, edge_index, logc, logr_c, gamma):
    xt = jnp.pad(x.T, ((0, 0), (0, NP - N)))
    pad = E_PAD - E
    src = jnp.concatenate([edge_index[0], jnp.zeros((pad,), jnp.int32)])
    dst = jnp.concatenate([edge_index[1], jnp.zeros((pad,), jnp.int32)])
    srows = src.reshape(E_PAD // 128, 128)
    drows = dst.reshape(E_PAD // 128, 128)
    logc16 = jnp.broadcast_to(logc, (L,)).astype(jnp.float32)
    logrc16 = jnp.broadcast_to(logr_c, (L,)).astype(jnp.float32)
    g16 = jnp.broadcast_to(gamma, (L,)).astype(jnp.float32)

    p = _edge_call(xt[0], xt[1], xt[2], srows, drows, logc16, logrc16)

    fp = _FLEN - 3 * NP
    f0 = jnp.pad(p[:3 * NP], (0, fp))
    f1 = jnp.pad(p[3 * NP:], (0, fp))
    vf = jnp.pad(jnp.pad(v.T, ((0, 0), (0, NP - N))).reshape(-1), (0, fp))

    out = _combine_call(f0, f1, vf, g16)
    return out[:3 * NP].reshape(3, NP)[:, :N].T


# speedup vs baseline: 82.8740x; 1.6045x over previous
import jax
import jax.numpy as jnp
from jax import lax
from jax.experimental import pallas as pl
from jax.experimental.pallas import tpu as pltpu
from jax.experimental.pallas import tpu_sc as plsc

N = 100000
NP = 100096
E = 6400000
NC = 2
NS = 16
NW = NC * NS
L = 16

CB = 2048
RC = CB // 128
CHUNKS = -(-(E // NW) // CB)
EW = CHUNKS * CB
E_PAD = NW * EW
ROWS_W = EW // 128
NPT = NP // NS

_CT = 9392
_FLEN = NW * _CT


def _edge_body(x0h, x1h, x2h, srows, drows, logc16, logrc16, p_out,
               x0, x1, x2, a0, a1, a2,
               sidx, didx, xs0, xs1, xs2, xd0, xd1, xd2, m0, m1, m2,
               stg, lcv, lrv, isem, g0sem, g1sem, ssem):
    cid = lax.axis_index("c")
    sid = lax.axis_index("s")
    wid = sid * NC + cid

    sl = pl.ds(sid * NPT, NPT)
    zeros = jnp.zeros((L,), jnp.float32)

    def zstg(i, _):
        stg[pl.ds(i * L, L)] = zeros
        return 0
    lax.fori_loop(0, NPT // L, zstg, 0)
    pltpu.sync_copy(stg, a0.at[sl])
    pltpu.sync_copy(stg, a1.at[sl])
    pltpu.sync_copy(stg, a2.at[sl])
    pltpu.sync_copy(x0h.at[sl], stg)
    pltpu.sync_copy(stg, x0.at[sl])
    pltpu.sync_copy(x1h.at[sl], stg)
    pltpu.sync_copy(stg, x1.at[sl])
    pltpu.sync_copy(x2h.at[sl], stg)
    pltpu.sync_copy(stg, x2.at[sl])
    pltpu.sync_copy(logc16, lcv)
    pltpu.sync_copy(logrc16, lrv)
    plsc.subcore_barrier()

    c2 = -2.0 * jnp.exp(lcv[...])
    rc = jnp.exp(lrv[...])

    row0 = wid * ROWS_W

    pltpu.sync_copy(srows.at[pl.ds(row0, RC)], sidx.at[pl.ds(0, RC)])
    pltpu.sync_copy(drows.at[pl.ds(row0, RC)], didx.at[pl.ds(0, RC)])

    def chunk(ci, _):
        slot = lax.rem(ci, 2)
        irow = slot * RC
        gsems = (g0sem, g1sem)

        def fire(j):
            d = pl.ds(j * 128, 128)
            sem = gsems[j % 2]
            return [
                pltpu.async_copy(x0.at[sidx.at[irow + j]], xs0.at[d], sem),
                pltpu.async_copy(x1.at[sidx.at[irow + j]], xs1.at[d], sem),
                pltpu.async_copy(x2.at[sidx.at[irow + j]], xs2.at[d], sem),
                pltpu.async_copy(x0.at[didx.at[irow + j]], xd0.at[d], sem),
                pltpu.async_copy(x1.at[didx.at[irow + j]], xd1.at[d], sem),
                pltpu.async_copy(x2.at[didx.at[irow + j]], xd2.at[d], sem),
            ]

        @pl.when(ci < CHUNKS - 1)
        def _():
            nrow = (1 - slot) * RC
            r = row0 + (ci + 1) * RC
            pltpu.async_copy(srows.at[pl.ds(r, RC)], sidx.at[pl.ds(nrow, RC)],
                             isem)
            pltpu.async_copy(drows.at[pl.ds(r, RC)], didx.at[pl.ds(nrow, RC)],
                             isem)

        gath = {0: fire(0), 1: fire(1)}
        scat = []

        def mk_group(jbase):
            def group(g, _):
                s = pl.ds(jbase + g * L, L)
                compute(s)
                return 0
            return group

        def compute(s):
            dr0 = xd0[s] - xs0[s]
            dr1 = xd1[s] - xs1[s]
            dr2 = xd2[s] - xs2[s]
            r2 = jnp.maximum(dr0 * dr0 + dr1 * dr1 + dr2 * dr2, 1e-26)
            t = r2 * jnp.float32(4.0 ** 45)
            sc = jnp.zeros((L,), jnp.float32) + jnp.float32(2.0 ** 45)
            for e in (32, 16, 8, 4, 2, 1):
                cnd = t >= jnp.float32(4.0 ** e)
                t = jnp.where(cnd, t * jnp.float32(4.0 ** -e), t)
                sc = jnp.where(cnd, sc * jnp.float32(2.0 ** -e), sc)
            y = 1.39518 + (-0.45231 + 0.05713 * t) * t
            y = y * (1.5 - 0.5 * t * y * y)
            y = y * (1.5 - 0.5 * t * y * y)
            y = y * (1.5 - 0.5 * t * y * y)
            y = sc * y
            absdr = r2 * y
            yc = jnp.minimum(y, 1e12)
            w = (c2 * (absdr - rc)) * yc
            m0[s] = w * dr0
            m1[s] = w * dr1
            m2[s] = w * dr2

        for j in range(RC):
            for cp in gath.pop(j):
                cp.wait()
            lax.fori_loop(0, 128 // L, mk_group(j * 128), 0)
            d = pl.ds(j * 128, 128)
            scat += [
                pltpu.async_copy(m0.at[d], a0.at[didx.at[irow + j]],
                                 ssem, add=True),
                pltpu.async_copy(m1.at[d], a1.at[didx.at[irow + j]],
                                 ssem, add=True),
                pltpu.async_copy(m2.at[d], a2.at[didx.at[irow + j]],
                                 ssem, add=True),
            ]
            if j + 2 < RC:
                gath[j + 2] = fire(j + 2)
        for cp in scat:
            cp.wait()

        @pl.when(ci < CHUNKS - 1)
        def _():
            nrow = (1 - slot) * RC
            pltpu.make_async_copy(srows.at[pl.ds(row0, RC)],
                                  sidx.at[pl.ds(nrow, RC)], isem).wait()
            pltpu.make_async_copy(drows.at[pl.ds(row0, RC)],
                                  didx.at[pl.ds(nrow, RC)], isem).wait()
        return 0

    lax.fori_loop(0, CHUNKS, chunk, 0)
    plsc.subcore_barrier()
    base = cid * 3 * NP + sid * NPT
    pltpu.sync_copy(a0.at[sl], stg)
    pltpu.sync_copy(stg, p_out.at[pl.ds(base, NPT)])
    pltpu.sync_copy(a1.at[sl], stg)
    pltpu.sync_copy(stg, p_out.at[pl.ds(base + NP, NPT)])
    pltpu.sync_copy(a2.at[sl], stg)
    pltpu.sync_copy(stg, p_out.at[pl.ds(base + 2 * NP, NPT)])


def _combine_body(f0, f1, vf, g16, out, b0, b1, bv, gv):
    cid = lax.axis_index("c")
    sid = lax.axis_index("s")
    wid = sid * NC + cid
    base = wid * _CT
    pltpu.sync_copy(f0.at[pl.ds(base, _CT)], b0)
    pltpu.sync_copy(f1.at[pl.ds(base, _CT)], b1)
    pltpu.sync_copy(vf.at[pl.ds(base, _CT)], bv)
    pltpu.sync_copy(g16, gv)
    g = gv[...]

    def step(i, _):
        s = pl.ds(i * L, L)
        b0[s] = b0[s] + b1[s] - g * bv[s]
        return 0
    lax.fori_loop(0, _CT // L, step, 0)
    pltpu.sync_copy(b0, out.at[pl.ds(base, _CT)])


_MESH = plsc.VectorSubcoreMesh(core_axis_name="c", subcore_axis_name="s")

_edge_call = pl.kernel(
    _edge_body,
    out_type=jax.ShapeDtypeStruct((6 * NP,), jnp.float32),
    mesh=_MESH,
    scratch_types=[
        pltpu.VMEM_SHARED((NP,), jnp.float32),
        pltpu.VMEM_SHARED((NP,), jnp.float32),
        pltpu.VMEM_SHARED((NP,), jnp.float32),
        pltpu.VMEM_SHARED((NP,), jnp.float32),
        pltpu.VMEM_SHARED((NP,), jnp.float32),
        pltpu.VMEM_SHARED((NP,), jnp.float32),
        pltpu.VMEM((2 * RC, 128), jnp.int32),
        pltpu.VMEM((2 * RC, 128), jnp.int32),
        pltpu.VMEM((CB,), jnp.float32),
        pltpu.VMEM((CB,), jnp.float32),
        pltpu.VMEM((CB,), jnp.float32),
        pltpu.VMEM((CB,), jnp.float32),
        pltpu.VMEM((CB,), jnp.float32),
        pltpu.VMEM((CB,), jnp.float32),
        pltpu.VMEM((CB,), jnp.float32),
        pltpu.VMEM((CB,), jnp.float32),
        pltpu.VMEM((CB,), jnp.float32),
        pltpu.VMEM((NPT,), jnp.float32),
        pltpu.VMEM((L,), jnp.float32),
        pltpu.VMEM((L,), jnp.float32),
        pltpu.SemaphoreType.DMA,
        pltpu.SemaphoreType.DMA,
        pltpu.SemaphoreType.DMA,
        pltpu.SemaphoreType.DMA,
    ],
)

_combine_call = pl.kernel(
    _combine_body,
    out_type=jax.ShapeDtypeStruct((_FLEN,), jnp.float32),
    mesh=_MESH,
    scratch_types=[
        pltpu.VMEM((_CT,), jnp.float32),
        pltpu.VMEM((_CT,), jnp.float32),
        pltpu.VMEM((_CT,), jnp.float32),
        pltpu.VMEM((L,), jnp.float32),
    ],
)


@jax.jit
def kernel(x, v, edge_index, logc, logr_c, gamma):
    xt = jnp.pad(x.T, ((0, 0), (0, NP - N)))
    pad = E_PAD - E
    src = jnp.concatenate([edge_index[0], jnp.zeros((pad,), jnp.int32)])
    dst = jnp.concatenate([edge_index[1], jnp.zeros((pad,), jnp.int32)])
    srows = src.reshape(E_PAD // 128, 128)
    drows = dst.reshape(E_PAD // 128, 128)
    logc16 = jnp.broadcast_to(logc, (L,)).astype(jnp.float32)
    logrc16 = jnp.broadcast_to(logr_c, (L,)).astype(jnp.float32)
    g16 = jnp.broadcast_to(gamma, (L,)).astype(jnp.float32)

    p = _edge_call(xt[0], xt[1], xt[2], srows, drows, logc16, logrc16)

    fp = _FLEN - 3 * NP
    f0 = jnp.pad(p[:3 * NP], (0, fp))
    f1 = jnp.pad(p[3 * NP:], (0, fp))
    vf = jnp.pad(jnp.pad(v.T, ((0, 0), (0, NP - N))).reshape(-1), (0, fp))

    out = _combine_call(f0, f1, vf, g16)
    return out[:3 * NP].reshape(3, NP)[:, :N].T
